# baseline (speedup 1.0000x reference)
import functools

import jax
import jax.numpy as jnp
from jax import lax
from jax.experimental import pallas as pl
from jax.experimental.pallas import tpu as pltpu
from jax.experimental.pallas import tpu_sc as plsc

T = 2048
H = 1024
FFN = 2048
E = 8
K = 2
EPAD = 128
S = K * T
SB = 512
BM = 256
NB = S // BM + E
RPAD = NB * BM
NTILES = 32
RPT = RPAD // NTILES
TPT = T // NTILES


def _router_body(hs_ref, gw_ref, logits_ref, wpair_ref):
    x = hs_ref[...]
    l = jax.lax.dot_general(
        x, gw_ref[...], (((1,), (1,)), ((), ())),
        preferred_element_type=jnp.float32)
    logits_ref[...] = l
    lane = jax.lax.broadcasted_iota(jnp.int32, l.shape, 1)
    neg = jnp.float32(-1e30)
    big = jnp.int32(10**9)
    lm = jnp.where(lane < E, l, neg)
    m1 = jnp.max(lm, axis=1, keepdims=True)
    id1 = jnp.min(jnp.where(lm == m1, lane, big), axis=1, keepdims=True)
    lm2 = jnp.where(lane == id1, neg, lm)
    m2 = jnp.max(lm2, axis=1, keepdims=True)
    id2 = jnp.min(jnp.where(lm2 == m2, lane, big), axis=1, keepdims=True)
    w1p = 1.0 / (1.0 + jnp.exp(m2 - m1))
    w2p = 1.0 - w1p
    wpair_ref[...] = jnp.where(
        lane == 0, w1p,
        jnp.where(lane == 1, w2p,
                  jnp.where(lane == 2, id1.astype(jnp.float32),
                            jnp.where(lane == 3, id2.astype(jnp.float32),
                                      0.0))))


def _dispatch_body(wpair_ref, combo_ref, counts_ref, acc_ref):
    b = pl.program_id(0)
    k = b // (T // SB)
    wp = wpair_ref[...]
    lane = jax.lax.broadcasted_iota(jnp.int32, wp.shape, 1)
    idc = jnp.sum(jnp.where(lane == 2 + k, wp, 0.0), axis=1, keepdims=True)
    sel = idc.astype(jnp.int32)
    wcol = jnp.sum(jnp.where(lane == k, wp, 0.0), axis=1, keepdims=True)
    mask = (lane == sel).astype(jnp.float32)
    ri = jax.lax.broadcasted_iota(jnp.int32, (SB, SB), 0)
    ci = jax.lax.broadcasted_iota(jnp.int32, (SB, SB), 1)
    tri = (ci < ri).astype(jnp.float32)
    rank = jax.lax.dot_general(
        tri, mask, (((1,), (0,)), ((), ())),
        preferred_element_type=jnp.float32)
    prev = jnp.where(b == 0, jnp.zeros_like(acc_ref[...]), acc_ref[...])
    rank = rank + prev
    rksel = jnp.sum(mask * rank, axis=1, keepdims=True)
    combo_ref[...] = jnp.where(
        lane == 0, rksel,
        jnp.where(lane == 1, idc, jnp.where(lane == 2, wcol, 0.0)))
    total = prev + jnp.sum(mask, axis=0, keepdims=True)
    acc_ref[...] = total

    @pl.when(b == pl.num_programs(0) - 1)
    def _():
        counts_ref[...] = jnp.broadcast_to(total, (8, EPAD))


def _pos_body(combo_ref, offs_ref, posw_ref):
    cb = combo_ref[...]
    lane = jax.lax.broadcasted_iota(jnp.int32, cb.shape, 1)
    rk = jnp.sum(jnp.where(lane == 0, cb, 0.0), axis=1, keepdims=True)
    idc = jnp.sum(jnp.where(lane == 1, cb, 0.0), axis=1, keepdims=True)
    wcol = jnp.sum(jnp.where(lane == 2, cb, 0.0), axis=1, keepdims=True)
    mask = (lane == idc.astype(jnp.int32)).astype(jnp.float32)
    off = jnp.sum(mask * offs_ref[0:1, :], axis=1, keepdims=True)
    pos = rk + off
    posw_ref[...] = jnp.where(lane == 0, pos,
                              jnp.where(lane == 1, wcol, 0.0))


def _sc_gather_body(pos_hbm, w_hbm, zi_hbm, zf_hbm, hs_hbm,
                    xs_hbm, rw_hbm,
                    posv, wv, ssv, rwv, b0, b1, g0, g1, w0, w1):
    c = lax.axis_index("c")
    s = lax.axis_index("s")
    wid = s * 2 + c
    pltpu.sync_copy(pos_hbm, posv)
    pltpu.sync_copy(w_hbm, wv)
    pltpu.sync_copy(zi_hbm, ssv)
    pltpu.sync_copy(zf_hbm, rwv)

    def body(i, carry):
        sl = pl.ds(i * 16, 16)
        pos = posv[sl]
        w = wv[sl]
        slot = lax.iota(jnp.int32, 16) + i * 16
        tok = slot & (T - 1)
        plsc.store_scatter(ssv, [pos], tok)
        plsc.store_scatter(rwv, [pos], w)
        return carry

    lax.fori_loop(0, S // 16, body, 0)

    @pl.when(wid == 0)
    def _():
        pltpu.sync_copy(rwv, rw_hbm)

    base = wid * RPT
    CH = 32
    NCK = RPT // CH
    bufs = (b0, b1)
    gsem = (g0, g1)
    wsem = (w0, w1)

    def gather(ck, buf, sem):
        r0 = base + ck * CH
        return pltpu.async_copy(hs_hbm.at[ssv.at[pl.ds(r0, CH)]], buf, sem)

    gds = [None, None]
    wds = [None, None]
    gds[0] = gather(0, bufs[0], gsem[0])
    for ck in range(NCK):
        cur = ck % 2
        nxt = (ck + 1) % 2
        if ck + 1 < NCK:
            if ck + 1 >= 2:
                wds[nxt].wait()
            gds[nxt] = gather(ck + 1, bufs[nxt], gsem[nxt])
        gds[cur].wait()
        wds[cur] = pltpu.async_copy(
            bufs[cur], xs_hbm.at[pl.ds(base + ck * CH, CH)], wsem[cur])
    wds[0].wait()
    wds[1].wait()


def _ffn_body(be_ref, nu_ref, x_ref, rw_ref, w1_ref, w3_ref, w2_ref, out_ref):
    b = pl.program_id(0)

    @pl.when(b < nu_ref[0])
    def _():
        x = x_ref[...]
        h1 = jax.lax.dot_general(
            x, w1_ref[0], (((1,), (1,)), ((), ())),
            preferred_element_type=jnp.float32)
        h3 = jax.lax.dot_general(
            x, w3_ref[0], (((1,), (1,)), ((), ())),
            preferred_element_type=jnp.float32)
        inter = (h1 * jax.nn.sigmoid(h1)) * h3
        y = jax.lax.dot_general(
            inter.astype(jnp.bfloat16), w2_ref[0], (((1,), (1,)), ((), ())),
            preferred_element_type=jnp.float32)
        out_ref[...] = y * rw_ref[...]


def _sc_combine_body(y_hbm, pos_hbm, out_hbm, p0v, p1v, y0b, y1b, sem0, sem1):
    c = lax.axis_index("c")
    s = lax.axis_index("s")
    wid = s * 2 + c
    base = wid * TPT
    pltpu.sync_copy(pos_hbm.at[pl.ds(base, TPT)], p0v)
    pltpu.sync_copy(pos_hbm.at[pl.ds(T + base, TPT)], p1v)
    for ck in range(TPT // 32):
        cp0 = pltpu.async_copy(
            y_hbm.at[p0v.at[pl.ds(ck * 32, 32)]], y0b, sem0)
        cp1 = pltpu.async_copy(
            y_hbm.at[p1v.at[pl.ds(ck * 32, 32)]], y1b, sem1)
        cp0.wait()
        cp1.wait()

        def body(i, carry):
            j = i >> 6
            sl = pl.ds((i & 63) * 16, 16)
            y0b[j, sl] = y0b[j, sl] + y1b[j, sl]
            return carry

        lax.fori_loop(0, 32 * (H // 16), body, 0)
        pltpu.sync_copy(y0b, out_hbm.at[pl.ds(base + ck * 32, 32)])


@functools.partial(jax.jit, static_argnums=())
def kernel(hidden_states, gate_w, w1, w3, w2):
    b, s, h = hidden_states.shape
    hs = hidden_states.reshape(-1, h)
    gw_pad = jnp.zeros((EPAD, H), jnp.float32).at[:E].set(gate_w)

    logits_pad, wpair = pl.pallas_call(
        _router_body,
        out_shape=(
            jax.ShapeDtypeStruct((T, EPAD), jnp.float32),
            jax.ShapeDtypeStruct((T, EPAD), jnp.float32),
        ),
    )(hs, gw_pad)

    combo, counts = pl.pallas_call(
        _dispatch_body,
        grid=(S // SB,),
        in_specs=[pl.BlockSpec((SB, EPAD), lambda b: (b % (T // SB), 0))],
        out_specs=(
            pl.BlockSpec((SB, EPAD), lambda b: (b, 0)),
            pl.BlockSpec((8, EPAD), lambda b: (0, 0)),
        ),
        out_shape=(
            jax.ShapeDtypeStruct((S, EPAD), jnp.float32),
            jax.ShapeDtypeStruct((8, EPAD), jnp.float32),
        ),
        scratch_shapes=[pltpu.VMEM((1, EPAD), jnp.float32)],
        compiler_params=pltpu.CompilerParams(
            dimension_semantics=("arbitrary",)),
    )(wpair)

    counts8 = counts[0, :E].astype(jnp.int32)
    padded = ((counts8 + BM - 1) // BM) * BM
    offs_next = jnp.cumsum(padded)
    offs = offs_next - padded
    nused = offs_next[E - 1] // BM
    bb = jnp.minimum(jnp.arange(NB, dtype=jnp.int32), nused - 1) * BM
    be = jnp.searchsorted(offs_next, bb, side="right").astype(jnp.int32)
    nu = nused.reshape(1)
    offs_row = jnp.broadcast_to(
        jnp.pad(offs.astype(jnp.float32), (0, EPAD - E)), (8, EPAD))

    posw = pl.pallas_call(
        _pos_body,
        out_shape=jax.ShapeDtypeStruct((S, EPAD), jnp.float32),
    )(combo, offs_row)
    pos = posw[:, 0].astype(jnp.int32)
    w_flat = posw[:, 1]
    zi = jnp.zeros((RPAD,), jnp.int32)
    zf = jnp.zeros((RPAD,), jnp.float32)

    mesh = plsc.VectorSubcoreMesh(
        core_axis_name="c", subcore_axis_name="s", num_cores=2,
        num_subcores=16)
    hsb = jax.lax.bitcast_convert_type(
        hs.astype(jnp.bfloat16).reshape(T, H // 2, 2), jnp.int32)
    sc_gather = pl.kernel(
        _sc_gather_body,
        out_type=(
            jax.ShapeDtypeStruct((RPAD, H // 2), jnp.int32),
            jax.ShapeDtypeStruct((RPAD,), jnp.float32),
        ),
        mesh=mesh,
        scratch_types=[
            pltpu.VMEM((S,), jnp.int32),
            pltpu.VMEM((S,), jnp.float32),
            pltpu.VMEM((RPAD,), jnp.int32),
            pltpu.VMEM((RPAD,), jnp.float32),
            pltpu.VMEM((32, H // 2), jnp.int32),
            pltpu.VMEM((32, H // 2), jnp.int32),
            pltpu.SemaphoreType.DMA,
            pltpu.SemaphoreType.DMA,
            pltpu.SemaphoreType.DMA,
            pltpu.SemaphoreType.DMA,
        ],
        compiler_params=pltpu.CompilerParams(needs_layout_passes=False),
    )
    x_sorted, rw_sorted = sc_gather(pos, w_flat, zi, zf, hsb)
    x_sorted = jax.lax.bitcast_convert_type(
        x_sorted, jnp.bfloat16).reshape(RPAD, H)

    w1b = w1.astype(jnp.bfloat16)
    w3b = w3.astype(jnp.bfloat16)
    w2b = w2.astype(jnp.bfloat16)

    y_sorted = pl.pallas_call(
        _ffn_body,
        grid_spec=pltpu.PrefetchScalarGridSpec(
            num_scalar_prefetch=2,
            grid=(NB,),
            in_specs=[
                pl.BlockSpec((BM, H), lambda b, be, nu: (b, 0)),
                pl.BlockSpec((BM, 1), lambda b, be, nu: (b, 0)),
                pl.BlockSpec((1, FFN, H), lambda b, be, nu: (be[b], 0, 0)),
                pl.BlockSpec((1, FFN, H), lambda b, be, nu: (be[b], 0, 0)),
                pl.BlockSpec((1, H, FFN), lambda b, be, nu: (be[b], 0, 0)),
            ],
            out_specs=pl.BlockSpec((BM, H), lambda b, be, nu: (b, 0)),
        ),
        out_shape=jax.ShapeDtypeStruct((RPAD, H), jnp.float32),
        compiler_params=pltpu.CompilerParams(
            dimension_semantics=("arbitrary",)),
    )(be, nu, x_sorted, rw_sorted.reshape(RPAD, 1), w1b, w3b, w2b)

    sc_combine = pl.kernel(
        _sc_combine_body,
        out_type=jax.ShapeDtypeStruct((T, H), jnp.float32),
        mesh=mesh,
        scratch_types=[
            pltpu.VMEM((TPT,), jnp.int32),
            pltpu.VMEM((TPT,), jnp.int32),
            pltpu.VMEM((32, H), jnp.float32),
            pltpu.VMEM((32, H), jnp.float32),
            pltpu.SemaphoreType.DMA,
            pltpu.SemaphoreType.DMA,
        ],
        compiler_params=pltpu.CompilerParams(needs_layout_passes=False),
    )
    final = sc_combine(y_sorted, pos)

    return (final.reshape(b, s, h), logits_pad[:, :E])


# speedup vs baseline: 1.7228x; 1.7228x over previous
import functools

import jax
import jax.numpy as jnp
from jax import lax
from jax.experimental import pallas as pl
from jax.experimental.pallas import tpu as pltpu
from jax.experimental.pallas import tpu_sc as plsc

T = 2048
H = 1024
FFN = 2048
E = 8
K = 2
EPAD = 128
S = K * T
SB = 512
BM = 256
NB = S // BM + E
RPAD = NB * BM
NTILES = 32
RPT = RPAD // NTILES
TPT = T // NTILES


def _router_body(hs_ref, gw_ref, logits_ref, wpair_ref, hsp_ref):
    x = hs_ref[...]
    xb = x.astype(jnp.bfloat16)
    lo = jax.lax.bitcast_convert_type(
        xb[:, :H // 2], jnp.uint16).astype(jnp.uint32)
    hi = jax.lax.bitcast_convert_type(
        xb[:, H // 2:], jnp.uint16).astype(jnp.uint32)
    hsp_ref[...] = jax.lax.bitcast_convert_type(lo | (hi << 16), jnp.int32)
    l = jax.lax.dot_general(
        x, gw_ref[...], (((1,), (1,)), ((), ())),
        preferred_element_type=jnp.float32)
    logits_ref[...] = l
    lane = jax.lax.broadcasted_iota(jnp.int32, l.shape, 1)
    neg = jnp.float32(-1e30)
    big = jnp.int32(10**9)
    lm = jnp.where(lane < E, l, neg)
    m1 = jnp.max(lm, axis=1, keepdims=True)
    id1 = jnp.min(jnp.where(lm == m1, lane, big), axis=1, keepdims=True)
    lm2 = jnp.where(lane == id1, neg, lm)
    m2 = jnp.max(lm2, axis=1, keepdims=True)
    id2 = jnp.min(jnp.where(lm2 == m2, lane, big), axis=1, keepdims=True)
    w1p = 1.0 / (1.0 + jnp.exp(m2 - m1))
    w2p = 1.0 - w1p
    wpair_ref[...] = jnp.where(
        lane == 0, w1p,
        jnp.where(lane == 1, w2p,
                  jnp.where(lane == 2, id1.astype(jnp.float32),
                            jnp.where(lane == 3, id2.astype(jnp.float32),
                                      0.0))))


def _dispatch_body(wpair_ref, combo_ref, counts_ref, acc_ref):
    b = pl.program_id(0)
    k = b // (T // SB)
    wp = wpair_ref[...]
    lane = jax.lax.broadcasted_iota(jnp.int32, wp.shape, 1)
    idc = jnp.sum(jnp.where(lane == 2 + k, wp, 0.0), axis=1, keepdims=True)
    sel = idc.astype(jnp.int32)
    wcol = jnp.sum(jnp.where(lane == k, wp, 0.0), axis=1, keepdims=True)
    mask = (lane == sel).astype(jnp.float32)
    ri = jax.lax.broadcasted_iota(jnp.int32, (SB, SB), 0)
    ci = jax.lax.broadcasted_iota(jnp.int32, (SB, SB), 1)
    tri = (ci < ri).astype(jnp.float32)
    rank = jax.lax.dot_general(
        tri, mask, (((1,), (0,)), ((), ())),
        preferred_element_type=jnp.float32)
    prev = jnp.where(b == 0, jnp.zeros_like(acc_ref[...]), acc_ref[...])
    rank = rank + prev
    rksel = jnp.sum(mask * rank, axis=1, keepdims=True)
    combo_ref[...] = jnp.where(
        lane == 0, rksel,
        jnp.where(lane == 1, idc, jnp.where(lane == 2, wcol, 0.0)))
    total = prev + jnp.sum(mask, axis=0, keepdims=True)
    acc_ref[...] = total

    @pl.when(b == pl.num_programs(0) - 1)
    def _():
        counts_ref[...] = jnp.broadcast_to(total, (8, EPAD))


def _pos_body(combo_ref, offs_ref, posw_ref):
    cb = combo_ref[...]
    lane = jax.lax.broadcasted_iota(jnp.int32, cb.shape, 1)
    rk = jnp.sum(jnp.where(lane == 0, cb, 0.0), axis=1, keepdims=True)
    idc = jnp.sum(jnp.where(lane == 1, cb, 0.0), axis=1, keepdims=True)
    wcol = jnp.sum(jnp.where(lane == 2, cb, 0.0), axis=1, keepdims=True)
    mask = (lane == idc.astype(jnp.int32)).astype(jnp.float32)
    off = jnp.sum(mask * offs_ref[0:1, :], axis=1, keepdims=True)
    pos = rk + off
    posw_ref[...] = jnp.where(lane == 0, pos,
                              jnp.where(lane == 1, wcol, 0.0))


def _sc_gather_body(pos_hbm, w_hbm, zi_hbm, zf_hbm, nu_hbm, hs_hbm,
                    xs_hbm, rw_hbm,
                    posv, wv, ssv, rwv, nuv, b0, b1, b2, g0, g1, g2):
    c = lax.axis_index("c")
    s = lax.axis_index("s")
    wid = s * 2 + c
    pltpu.sync_copy(pos_hbm, posv)
    pltpu.sync_copy(w_hbm, wv)
    pltpu.sync_copy(zi_hbm, ssv)
    pltpu.sync_copy(zf_hbm, rwv)
    pltpu.sync_copy(nu_hbm, nuv)

    def body(i, carry):
        sl = pl.ds(i * 16, 16)
        pos = posv[sl]
        w = wv[sl]
        slot = lax.iota(jnp.int32, 16) + i * 16
        tok = slot & (T - 1)
        plsc.store_scatter(ssv, [pos], tok)
        plsc.store_scatter(rwv, [pos], w)
        return carry

    lax.fori_loop(0, S // 16, body, 0)

    @pl.when(wid == 0)
    def _():
        pltpu.sync_copy(rwv, rw_hbm)

    base = wid * RPT
    CH = RPT // 3
    nlim = nuv[pl.ds(0, 16)][0] * BM
    bufs = (b0, b1, b2)
    gsem = (g0, g1, g2)
    for ck in range(3):
        @pl.when(base + ck * CH < nlim)
        def _(ck=ck):
            pltpu.async_copy(
                hs_hbm.at[ssv.at[pl.ds(base + ck * CH, CH)]],
                bufs[ck], gsem[ck])
    for ck in range(3):
        @pl.when(base + ck * CH < nlim)
        def _(ck=ck):
            pltpu.make_async_copy(
                hs_hbm.at[ssv.at[pl.ds(base + ck * CH, CH)]],
                bufs[ck], gsem[ck]).wait()
            pltpu.sync_copy(bufs[ck], xs_hbm.at[pl.ds(base + ck * CH, CH)])


def _ffn_body(be_ref, nu_ref, x_ref, rw_ref, w1_ref, w3_ref, w2_ref, out_ref):
    b = pl.program_id(0)

    @pl.when(b < nu_ref[0])
    def _():
        xp = jax.lax.bitcast_convert_type(x_ref[...], jnp.uint32)
        lo = jax.lax.bitcast_convert_type(
            (xp & jnp.uint32(0xFFFF)).astype(jnp.uint16), jnp.bfloat16)
        hi = jax.lax.bitcast_convert_type(
            (xp >> 16).astype(jnp.uint16), jnp.bfloat16)
        x = jnp.concatenate([lo, hi], axis=1)
        h1 = jax.lax.dot_general(
            x, w1_ref[0], (((1,), (1,)), ((), ())),
            preferred_element_type=jnp.float32)
        h3 = jax.lax.dot_general(
            x, w3_ref[0], (((1,), (1,)), ((), ())),
            preferred_element_type=jnp.float32)
        inter = (h1 * jax.nn.sigmoid(h1)) * h3
        y = jax.lax.dot_general(
            inter.astype(jnp.bfloat16), w2_ref[0], (((1,), (1,)), ((), ())),
            preferred_element_type=jnp.float32)
        out_ref[...] = y * rw_ref[...]


def _sc_combine_body(y_hbm, pos_hbm, out_hbm, p0v, p1v, y0b, y1b, sem0, sem1):
    c = lax.axis_index("c")
    s = lax.axis_index("s")
    wid = s * 2 + c
    base = wid * TPT
    pltpu.sync_copy(pos_hbm.at[pl.ds(base, TPT)], p0v)
    pltpu.sync_copy(pos_hbm.at[pl.ds(T + base, TPT)], p1v)
    for ck in range(TPT // 32):
        cp0 = pltpu.async_copy(
            y_hbm.at[p0v.at[pl.ds(ck * 32, 32)]], y0b, sem0)
        cp1 = pltpu.async_copy(
            y_hbm.at[p1v.at[pl.ds(ck * 32, 32)]], y1b, sem1)
        cp0.wait()
        cp1.wait()

        def body(i, carry):
            j = i >> 6
            sl = pl.ds((i & 63) * 16, 16)
            y0b[j, sl] = y0b[j, sl] + y1b[j, sl]
            return carry

        lax.fori_loop(0, 32 * (H // 16), body, 0)
        pltpu.sync_copy(y0b, out_hbm.at[pl.ds(base + ck * 32, 32)])


@functools.partial(jax.jit, static_argnums=())
def kernel(hidden_states, gate_w, w1, w3, w2):
    b, s, h = hidden_states.shape
    hs = hidden_states.reshape(-1, h)
    gw_pad = jnp.zeros((EPAD, H), jnp.float32).at[:E].set(gate_w)

    logits_pad, wpair, hsp = pl.pallas_call(
        _router_body,
        out_shape=(
            jax.ShapeDtypeStruct((T, EPAD), jnp.float32),
            jax.ShapeDtypeStruct((T, EPAD), jnp.float32),
            jax.ShapeDtypeStruct((T, H // 2), jnp.int32),
        ),
    )(hs, gw_pad)

    combo, counts = pl.pallas_call(
        _dispatch_body,
        grid=(S // SB,),
        in_specs=[pl.BlockSpec((SB, EPAD), lambda b: (b % (T // SB), 0))],
        out_specs=(
            pl.BlockSpec((SB, EPAD), lambda b: (b, 0)),
            pl.BlockSpec((8, EPAD), lambda b: (0, 0)),
        ),
        out_shape=(
            jax.ShapeDtypeStruct((S, EPAD), jnp.float32),
            jax.ShapeDtypeStruct((8, EPAD), jnp.float32),
        ),
        scratch_shapes=[pltpu.VMEM((1, EPAD), jnp.float32)],
        compiler_params=pltpu.CompilerParams(
            dimension_semantics=("arbitrary",)),
    )(wpair)

    counts8 = counts[0, :E].astype(jnp.int32)
    padded = ((counts8 + BM - 1) // BM) * BM
    offs_next = jnp.cumsum(padded)
    offs = offs_next - padded
    nused = offs_next[E - 1] // BM
    bb = jnp.minimum(jnp.arange(NB, dtype=jnp.int32), nused - 1) * BM
    be = jnp.searchsorted(offs_next, bb, side="right").astype(jnp.int32)
    nu = nused.reshape(1)
    offs_row = jnp.broadcast_to(
        jnp.pad(offs.astype(jnp.float32), (0, EPAD - E)), (8, EPAD))

    posw = pl.pallas_call(
        _pos_body,
        out_shape=jax.ShapeDtypeStruct((S, EPAD), jnp.float32),
    )(combo, offs_row)
    pos = posw[:, 0].astype(jnp.int32)
    w_flat = posw[:, 1]
    zi = jnp.zeros((RPAD,), jnp.int32)
    zf = jnp.zeros((RPAD,), jnp.float32)

    mesh = plsc.VectorSubcoreMesh(
        core_axis_name="c", subcore_axis_name="s", num_cores=2,
        num_subcores=16)
    nu16 = jnp.zeros((16,), jnp.int32).at[0].set(nused)
    sc_gather = pl.kernel(
        _sc_gather_body,
        out_type=(
            jax.ShapeDtypeStruct((RPAD, H // 2), jnp.int32),
            jax.ShapeDtypeStruct((RPAD,), jnp.float32),
        ),
        mesh=mesh,
        scratch_types=[
            pltpu.VMEM((S,), jnp.int32),
            pltpu.VMEM((S,), jnp.float32),
            pltpu.VMEM((RPAD,), jnp.int32),
            pltpu.VMEM((RPAD,), jnp.float32),
            pltpu.VMEM((16,), jnp.int32),
            pltpu.VMEM((RPT // 3, H // 2), jnp.int32),
            pltpu.VMEM((RPT // 3, H // 2), jnp.int32),
            pltpu.VMEM((RPT // 3, H // 2), jnp.int32),
            pltpu.SemaphoreType.DMA,
            pltpu.SemaphoreType.DMA,
            pltpu.SemaphoreType.DMA,
        ],
        compiler_params=pltpu.CompilerParams(needs_layout_passes=False),
    )
    x_sorted, rw_sorted = sc_gather(pos, w_flat, zi, zf, nu16, hsp)

    w1b = w1.astype(jnp.bfloat16)
    w3b = w3.astype(jnp.bfloat16)
    w2b = w2.astype(jnp.bfloat16)

    y_sorted = pl.pallas_call(
        _ffn_body,
        grid_spec=pltpu.PrefetchScalarGridSpec(
            num_scalar_prefetch=2,
            grid=(NB,),
            in_specs=[
                pl.BlockSpec((BM, H // 2), lambda b, be, nu: (b, 0)),
                pl.BlockSpec((BM, 1), lambda b, be, nu: (b, 0)),
                pl.BlockSpec((1, FFN, H), lambda b, be, nu: (be[b], 0, 0)),
                pl.BlockSpec((1, FFN, H), lambda b, be, nu: (be[b], 0, 0)),
                pl.BlockSpec((1, H, FFN), lambda b, be, nu: (be[b], 0, 0)),
            ],
            out_specs=pl.BlockSpec((BM, H), lambda b, be, nu: (b, 0)),
        ),
        out_shape=jax.ShapeDtypeStruct((RPAD, H), jnp.float32),
        compiler_params=pltpu.CompilerParams(
            dimension_semantics=("arbitrary",)),
    )(be, nu, x_sorted, rw_sorted.reshape(RPAD, 1), w1b, w3b, w2b)

    sc_combine = pl.kernel(
        _sc_combine_body,
        out_type=jax.ShapeDtypeStruct((T, H), jnp.float32),
        mesh=mesh,
        scratch_types=[
            pltpu.VMEM((TPT,), jnp.int32),
            pltpu.VMEM((TPT,), jnp.int32),
            pltpu.VMEM((32, H), jnp.float32),
            pltpu.VMEM((32, H), jnp.float32),
            pltpu.SemaphoreType.DMA,
            pltpu.SemaphoreType.DMA,
        ],
        compiler_params=pltpu.CompilerParams(needs_layout_passes=False),
    )
    final = sc_combine(y_sorted, pos)

    return (final.reshape(b, s, h), logits_pad[:, :E])
